# baseline (speedup 1.0000x reference)
import functools

import jax
import jax.numpy as jnp
from jax import lax
from jax.experimental import pallas as pl
from jax.experimental.pallas import tpu as pltpu
from jax.experimental.pallas import tpu_sc as plsc

NUM_CORES = 2
NUM_SUBCORES = 16
NUM_WORKERS = NUM_CORES * NUM_SUBCORES
CHUNK = 64
NBUF = 5
GDEPTH = 2


@functools.lru_cache(maxsize=None)
def _build_lookup(n_chunks, vocab, dim):
    b_per_w = n_chunks * CHUNK
    total = NUM_WORKERS * b_per_w
    mesh = plsc.VectorSubcoreMesh(
        core_axis_name="c",
        subcore_axis_name="s",
        num_cores=NUM_CORES,
        num_subcores=NUM_SUBCORES,
    )
    lag = NBUF - GDEPTH
    assert n_chunks % NBUF == 0 and n_chunks >= NBUF and 0 < lag < NBUF

    @functools.partial(
        pl.kernel,
        out_type=jax.ShapeDtypeStruct((total, dim), jnp.float32),
        mesh=mesh,
        scratch_types=[
            pltpu.VMEM((n_chunks, CHUNK), jnp.int32),
            pltpu.VMEM((NBUF, CHUNK, dim), jnp.float32),
            pltpu.SemaphoreType.DMA,
            pltpu.SemaphoreType.DMA,
        ],
    )
    def lookup(idx_hbm, table_hbm, out_hbm, idx_v, rows_v, gsem, osem):
        wid = lax.axis_index("s") * NUM_CORES + lax.axis_index("c")
        base = wid * b_per_w
        pltpu.sync_copy(idx_hbm.at[wid], idx_v)

        def gather(j, slot):
            pltpu.async_copy(table_hbm.at[idx_v.at[j]], rows_v.at[slot], gsem)

        def wait_gather(slot):
            pltpu.make_async_copy(
                table_hbm.at[idx_v.at[0]], rows_v.at[slot], gsem
            ).wait()

        def put(j, slot):
            pltpu.async_copy(
                rows_v.at[slot], out_hbm.at[pl.ds(base + j * CHUNK, CHUNK)], osem
            )

        def wait_put(slot):
            pltpu.make_async_copy(
                rows_v.at[slot], out_hbm.at[pl.ds(base, CHUNK)], osem
            ).wait()

        for j in range(GDEPTH):
            gather(j, j)

        @pl.loop(0, n_chunks, step=NBUF)
        def _(j0):
            for b in range(NBUF):
                j = j0 + b
                nxt = (b + GDEPTH) % NBUF
                wait_gather(b)
                @pl.when(j >= lag)
                def _():
                    wait_put(nxt)

                @pl.when(j + GDEPTH < n_chunks)
                def _():
                    gather(j + GDEPTH, nxt)

                put(j, b)

        for _ in range(lag):
            wait_put(0)

    return lookup


def kernel(action_indices, table):
    n, k = action_indices.shape
    vocab, dim = table.shape
    total = n * k
    assert total % (NUM_WORKERS * CHUNK) == 0
    n_chunks = total // (NUM_WORKERS * CHUNK)
    idx = action_indices.astype(jnp.int32).T.reshape(NUM_WORKERS, n_chunks, CHUNK)
    out = _build_lookup(n_chunks, vocab, dim)(idx, table)
    return out.reshape(k, n, dim).transpose(1, 0, 2)


# speedup vs baseline: 1.0095x; 1.0095x over previous
import functools

import jax
import jax.numpy as jnp
from jax import lax
from jax.experimental import pallas as pl
from jax.experimental.pallas import tpu as pltpu
from jax.experimental.pallas import tpu_sc as plsc

NUM_CORES = 2
NUM_SUBCORES = 16
NUM_WORKERS = NUM_CORES * NUM_SUBCORES
CHUNK = 80
NBUF = 4
GDEPTH = 2


@functools.lru_cache(maxsize=None)
def _build_lookup(n_chunks, vocab, dim):
    b_per_w = n_chunks * CHUNK
    total = NUM_WORKERS * b_per_w
    mesh = plsc.VectorSubcoreMesh(
        core_axis_name="c",
        subcore_axis_name="s",
        num_cores=NUM_CORES,
        num_subcores=NUM_SUBCORES,
    )
    lag = NBUF - GDEPTH
    assert n_chunks % NBUF == 0 and n_chunks >= NBUF and 0 < lag < NBUF

    @functools.partial(
        pl.kernel,
        out_type=jax.ShapeDtypeStruct((total, dim), jnp.float32),
        mesh=mesh,
        scratch_types=[
            pltpu.VMEM((n_chunks, CHUNK), jnp.int32),
            pltpu.VMEM((NBUF, CHUNK, dim), jnp.float32),
            pltpu.SemaphoreType.DMA,
            pltpu.SemaphoreType.DMA,
        ],
    )
    def lookup(idx_hbm, table_hbm, out_hbm, idx_v, rows_v, gsem, osem):
        wid = lax.axis_index("s") * NUM_CORES + lax.axis_index("c")
        base = wid * b_per_w
        pltpu.sync_copy(idx_hbm.at[wid], idx_v)

        def gather(j, slot):
            pltpu.async_copy(table_hbm.at[idx_v.at[j]], rows_v.at[slot], gsem)

        def wait_gather(slot):
            pltpu.make_async_copy(
                table_hbm.at[idx_v.at[0]], rows_v.at[slot], gsem
            ).wait()

        def put(j, slot):
            pltpu.async_copy(
                rows_v.at[slot], out_hbm.at[pl.ds(base + j * CHUNK, CHUNK)], osem
            )

        def wait_put(slot):
            pltpu.make_async_copy(
                rows_v.at[slot], out_hbm.at[pl.ds(base, CHUNK)], osem
            ).wait()

        for j in range(GDEPTH):
            gather(j, j)

        @pl.loop(0, n_chunks, step=NBUF)
        def _(j0):
            for b in range(NBUF):
                j = j0 + b
                nxt = (b + GDEPTH) % NBUF
                wait_gather(b)
                @pl.when(j >= lag)
                def _():
                    wait_put(nxt)

                @pl.when(j + GDEPTH < n_chunks)
                def _():
                    gather(j + GDEPTH, nxt)

                put(j, b)

        for _ in range(lag):
            wait_put(0)

    return lookup


def kernel(action_indices, table):
    n, k = action_indices.shape
    vocab, dim = table.shape
    total = n * k
    assert total % (NUM_WORKERS * CHUNK) == 0
    n_chunks = total // (NUM_WORKERS * CHUNK)
    idx = action_indices.astype(jnp.int32).T.reshape(NUM_WORKERS, n_chunks, CHUNK)
    out = _build_lookup(n_chunks, vocab, dim)(idx, table)
    return out.reshape(k, n, dim).transpose(1, 0, 2)
